# Initial kernel scaffold; baseline (speedup 1.0000x reference)
import jax
import jax.numpy as jnp
from jax.experimental import pallas as pl


def kernel(inputs, W):
    raise NotImplementedError("write your pallas kernel here")


# speedup vs baseline: 1.0014x; 1.0014x over previous
import functools

import jax
import jax.numpy as jnp
from jax import lax
from jax.experimental import pallas as pl
from jax.experimental.pallas import tpu as pltpu
from jax.experimental.pallas import tpu_sc as plsc

NUM_CODES = 8192
DIM = 256
NUM_TOKENS = 16384
TOK_BLK = 256
N_BLKS = NUM_TOKENS // TOK_BLK
_CHUNK = 2736


def _argmin_onehot_body(z_ref, sz_ref, sw_ref, wt_ref, oh_ref, idx_ref, cnt_ref):
    zb = z_ref[...].astype(jnp.bfloat16)
    m = jnp.dot(zb, wt_ref[...], preferred_element_type=jnp.float32)
    d = (sz_ref[...] + sw_ref[...]) - 2.0 * m
    iota = lax.broadcasted_iota(jnp.int32, (TOK_BLK, NUM_CODES), 1)
    accv = None
    acci = None
    for s in range(0, NUM_CODES, _CHUNK):
        e = min(s + _CHUNK, NUM_CODES)
        mask = (iota >= s) & (iota < e)
        dm = jnp.where(mask, d, jnp.float32(jnp.inf))
        vc = jnp.min(dm, axis=1)
        ic = jnp.min(jnp.where(dm == vc[:, None], iota, jnp.int32(NUM_CODES)), axis=1)
        if accv is None:
            accv, acci = vc, ic
        else:
            lt = vc < accv
            take = lt | ((vc == accv) & (ic < acci))
            accv = jnp.where(lt, vc, accv)
            acci = jnp.where(take, ic, acci)
        accv = accv.astype(jnp.bfloat16).astype(jnp.float32)
    idx = acci
    oh = (iota == idx[:, None]).astype(jnp.float32)
    oh_ref[...] = oh
    idx_ref[...] = idx
    colsum = jnp.sum(oh, axis=0)[None, :]

    @pl.when(pl.program_id(0) == 0)
    def _init():
        cnt_ref[...] = colsum

    @pl.when(pl.program_id(0) != 0)
    def _acc():
        cnt_ref[...] += colsum


def _run_argmin_onehot(flat, sz, sw, wt_bf):
    return pl.pallas_call(
        _argmin_onehot_body,
        grid=(N_BLKS,),
        in_specs=[
            pl.BlockSpec((TOK_BLK, DIM), lambda i: (i, 0)),
            pl.BlockSpec((TOK_BLK, 1), lambda i: (i, 0)),
            pl.BlockSpec((1, NUM_CODES), lambda i: (0, 0)),
            pl.BlockSpec((DIM, NUM_CODES), lambda i: (0, 0)),
        ],
        out_specs=[
            pl.BlockSpec((TOK_BLK, NUM_CODES), lambda i: (i, 0)),
            pl.BlockSpec((TOK_BLK,), lambda i: (i,)),
            pl.BlockSpec((1, NUM_CODES), lambda i: (0, 0)),
        ],
        out_shape=[
            jax.ShapeDtypeStruct((NUM_TOKENS, NUM_CODES), jnp.float32),
            jax.ShapeDtypeStruct((NUM_TOKENS,), jnp.int32),
            jax.ShapeDtypeStruct((1, NUM_CODES), jnp.float32),
        ],
    )(flat, sz, sw, wt_bf)


_SC_ROWS_PER_TILE = NUM_TOKENS // 32
_SC_CHUNK = 256


def _run_gather(W, idx):
    mesh = plsc.VectorSubcoreMesh(core_axis_name="c", subcore_axis_name="s")

    @functools.partial(
        pl.kernel,
        mesh=mesh,
        out_type=jax.ShapeDtypeStruct((NUM_TOKENS, DIM), jnp.float32),
        scratch_types=[
            pltpu.VMEM((_SC_CHUNK,), jnp.int32),
            pltpu.VMEM((_SC_CHUNK, DIM), jnp.float32),
            pltpu.SemaphoreType.DMA,
        ],
    )
    def gather_kernel(w_hbm, idx_hbm, out_hbm, idx_v, rows_v, sem):
        wid = lax.axis_index("s") * 2 + lax.axis_index("c")
        base = wid * _SC_ROWS_PER_TILE
        for c in range(_SC_ROWS_PER_TILE // _SC_CHUNK):
            off = base + c * _SC_CHUNK
            pltpu.sync_copy(idx_hbm.at[pl.ds(off, _SC_CHUNK)], idx_v)
            pltpu.async_copy(w_hbm.at[idx_v], rows_v, sem).wait()
            pltpu.sync_copy(rows_v, out_hbm.at[pl.ds(off, _SC_CHUNK)])

    return gather_kernel(W, idx)


_FIN_BLK = 2048
_FIN_STEPS = NUM_TOKENS // _FIN_BLK


def _finalize_body(z_ref, q_ref, cnt_ref, st_ref, vq_ref, pp_ref, acc_ref):
    z = z_ref[...]
    q = q_ref[...]
    st_ref[...] = z + (q - z)
    blk_ssq = jnp.sum((q - z) ** 2)

    @pl.when(pl.program_id(0) == 0)
    def _init():
        acc_ref[0] = blk_ssq

    @pl.when(pl.program_id(0) != 0)
    def _acc():
        acc_ref[0] += blk_ssq

    @pl.when(pl.program_id(0) == _FIN_STEPS - 1)
    def _fin():
        vq_ref[...] = jnp.reshape(acc_ref[0] / jnp.float32(NUM_TOKENS * DIM), (1, 1))
        p = cnt_ref[...] * jnp.float32(1.0 / NUM_TOKENS)
        ent = jnp.sum(p * jnp.log(p + 1e-10))
        pp_ref[...] = jnp.reshape(jnp.exp(-ent), (1, 1))


def _run_finalize(flat, quantized, counts):
    return pl.pallas_call(
        _finalize_body,
        grid=(_FIN_STEPS,),
        in_specs=[
            pl.BlockSpec((_FIN_BLK, DIM), lambda i: (i, 0)),
            pl.BlockSpec((_FIN_BLK, DIM), lambda i: (i, 0)),
            pl.BlockSpec((1, NUM_CODES), lambda i: (0, 0)),
        ],
        out_specs=[
            pl.BlockSpec((_FIN_BLK, DIM), lambda i: (i, 0)),
            pl.BlockSpec((1, 1), lambda i: (0, 0)),
            pl.BlockSpec((1, 1), lambda i: (0, 0)),
        ],
        out_shape=[
            jax.ShapeDtypeStruct((NUM_TOKENS, DIM), jnp.float32),
            jax.ShapeDtypeStruct((1, 1), jnp.float32),
            jax.ShapeDtypeStruct((1, 1), jnp.float32),
        ],
        scratch_shapes=[pltpu.SMEM((1,), jnp.float32)],
    )(flat, quantized, counts)


def kernel(inputs, W):
    input_shape = inputs.shape
    flat = inputs.reshape(-1, DIM)
    sz = jnp.sum(flat ** 2, axis=1, keepdims=True)
    sw = jnp.sum(W ** 2, axis=1)[None, :]
    wt_bf = W.astype(jnp.bfloat16).T

    encodings, idx, counts = _run_argmin_onehot(flat, sz, sw, wt_bf)
    quantized = _run_gather(W, idx)
    st, vq, pp = _run_finalize(flat, quantized, counts)

    vq_loss = vq.reshape(())
    perplexity = pp.reshape(())
    return (
        st.reshape(input_shape),
        vq_loss,
        vq_loss,
        perplexity,
        encodings,
        idx.reshape(input_shape[:-1]),
    )
